# baseline (speedup 1.0000x reference)
import jax
import jax.numpy as jnp
from jax import lax
from jax.experimental import pallas as pl
from jax.experimental.pallas import tpu as pltpu
from jax.experimental.pallas import tpu_sc as plsc

_D = 32
_B = 16384
_TW = 128

_info = plsc.get_sparse_core_info()
_NC, _NS = _info.num_cores, _info.num_subcores
_NW = _NC * _NS
_BPW = _B // _NW
_NT = 8
_NJ = _BPW // (2 * _NT)


def _gather_body(table_hbm, idx_hbm, out_hbm, idx_v, tba, tbb, slab,
                 sema, semb):
    wid = lax.axis_index("s") * _NC + lax.axis_index("c")
    base = wid * _BPW
    pltpu.sync_copy(idx_hbm.at[pl.ds(pl.multiple_of(base, 8), _BPW)], idx_v)
    f_lo = lax.iota(jnp.int32, 16)
    f_hi = f_lo + 16

    def enqueue(tv, lo, tb, sem):
        for s in range(_NT):
            pltpu.make_async_copy(
                table_hbm.at[:, pl.ds(pl.multiple_of(tv[lo + s], _TW), _TW)],
                tb.at[s],
                sem,
            ).start()

    def drain_extract(cv, lo, k0, tb, sem):
        for s in range(_NT):
            pltpu.make_async_copy(
                table_hbm.at[:, pl.ds(0, _TW)], tb.at[s], sem
            ).wait()
        for s in range(_NT):
            c16 = jnp.full((16,), cv[lo + s], jnp.int32)
            a = plsc.load_gather(tb.at[s], [f_lo, c16])
            b = plsc.load_gather(tb.at[s], [f_hi, c16])
            k = k0 + s
            slab[pl.ds(k * _D, 16)] = a
            slab[pl.ds(k * _D + 16, 16)] = b

    v0 = idx_v[pl.ds(0, 16)]
    tv0 = lax.shift_left(lax.shift_right_logical(v0, 7), 7)
    enqueue(tv0, 0, tba, sema)
    enqueue(tv0, _NT, tbb, semb)

    def body(j, carry):
        v = idx_v[pl.ds(j * 16, 16)]
        cv = lax.bitwise_and(v, _TW - 1)
        drain_extract(cv, 0, j * 16, tba, sema)

        @pl.when(j < _NJ - 1)
        def _():
            vn = idx_v[pl.ds((j + 1) * 16, 16)]
            tvn = lax.shift_left(lax.shift_right_logical(vn, 7), 7)
            enqueue(tvn, 0, tba, sema)

        drain_extract(cv, _NT, j * 16 + _NT, tbb, semb)

        @pl.when(j < _NJ - 1)
        def _():
            vn = idx_v[pl.ds((j + 1) * 16, 16)]
            tvn = lax.shift_left(lax.shift_right_logical(vn, 7), 7)
            enqueue(tvn, _NT, tbb, semb)

        return carry

    lax.fori_loop(0, _NJ, body, 0)
    pltpu.sync_copy(
        slab, out_hbm.at[pl.ds(pl.multiple_of(base * _D, 8), _BPW * _D)]
    )


@jax.jit
def kernel(node_vecs, country_idx):
    table_t = node_vecs.T
    idx = country_idx.astype(jnp.int32).reshape(_B)
    mesh = plsc.VectorSubcoreMesh(core_axis_name="c", subcore_axis_name="s")
    f = pl.kernel(
        _gather_body,
        mesh=mesh,
        out_type=jax.ShapeDtypeStruct((_B * _D,), jnp.float32),
        scratch_types=[
            pltpu.VMEM((_BPW,), jnp.int32),
            pltpu.VMEM((_NT, _D, _TW), jnp.float32),
            pltpu.VMEM((_NT, _D, _TW), jnp.float32),
            pltpu.VMEM((_BPW * _D,), jnp.float32),
            pltpu.SemaphoreType.DMA,
            pltpu.SemaphoreType.DMA,
        ],
        compiler_params=pltpu.CompilerParams(
            skip_device_barrier=True,
            disable_semaphore_checks=True,
            disable_bounds_checks=True,
            needs_layout_passes=False,
        ),
    )
    return f(table_t, idx).reshape(_B, _D)
